# baseline (speedup 1.0000x reference)
import functools
import math

import jax
import jax.numpy as jnp
from jax import lax
from jax.experimental import pallas as pl
from jax.experimental.pallas import tpu as pltpu
from jax.experimental.pallas import tpu_sc as plsc

N = 10000
E = 320000
D = 128
D_OUT = 64
NCORES = 2
NSUB = 16
NW = NCORES * NSUB
EPW = E // NW
CHUNK = 128
PCH = (EPW + CHUNK - 1) // CHUNK
PCH += PCH % 2
EPW_PAD = PCH * CHUNK
NPAIR = PCH // 2
OUT_CHUNK = 80
NOUT_CHUNKS = N // OUT_CHUNK
SCALE = 1.0 / math.sqrt(128.0)

_MESH = plsc.VectorSubcoreMesh(
    core_axis_name="c", subcore_axis_name="s", num_cores=NCORES, num_subcores=NSUB
)


def _spmm_body(x_hbm, col_hbm, row_hbm, w_hbm, out_hbm,
               colv, rb0, rb1, wb0, wb1, g0, g1, acc,
               gsem0, gsem1, rsem0, rsem1, wsem0, wsem1, ssem0, ssem1):
    c = lax.axis_index("c")
    s = lax.axis_index("s")
    wid = c * NSUB + s

    pltpu.sync_copy(col_hbm.at[wid], colv)

    def zero_row(e, carry):
        for j in range(D // 16):
            g0[e, pl.ds(j * 16, 16)] = jnp.zeros((16,), jnp.float32)
        return carry

    lax.fori_loop(0, OUT_CHUNK, zero_row, 0)
    for i in range((NOUT_CHUNKS + NSUB - 1) // NSUB):
        g = s + i * NSUB

        @pl.when(g < NOUT_CHUNKS)
        def _():
            pltpu.sync_copy(
                g0.at[pl.ds(0, OUT_CHUNK)], acc.at[pl.ds(g * OUT_CHUNK, OUT_CHUNK)]
            )

    plsc.subcore_barrier()

    def _scale(gbuf, wb):
        def scale_group(g, carry2):
            b16 = pl.multiple_of(g * 16, 16)
            wgrp = wb[0, pl.ds(b16, 16)]
            for e16 in range(16):
                we = wgrp[e16]
                r = b16 + e16
                for j in range(D // 16):
                    gbuf[r, pl.ds(j * 16, 16)] = gbuf[r, pl.ds(j * 16, 16)] * we
            return carry2

        lax.fori_loop(0, CHUNK // 16, scale_group, 0)

    def _issue(k, gbuf, rb, wb, gsem, rsem, wsem):
        pltpu.async_copy(x_hbm.at[colv.at[k]], gbuf, gsem)
        pltpu.async_copy(row_hbm.at[wid, pl.ds(k, 1)], rb, rsem)
        pltpu.async_copy(w_hbm.at[wid, pl.ds(k, 1)], wb, wsem)

    def _drain(k, gbuf, rb, wb, gsem, rsem, wsem, ssem):
        pltpu.make_async_copy(x_hbm.at[colv.at[k]], gbuf, gsem).wait()
        pltpu.make_async_copy(row_hbm.at[wid, pl.ds(k, 1)], rb, rsem).wait()
        pltpu.make_async_copy(w_hbm.at[wid, pl.ds(k, 1)], wb, wsem).wait()
        _scale(gbuf, wb)
        pltpu.async_copy(gbuf, acc.at[rb.at[0]], ssem, add=True)

    def _swait(gbuf, rb, ssem):
        pltpu.make_async_copy(gbuf, acc.at[rb.at[0]], ssem).wait()

    _issue(0, g0, rb0, wb0, gsem0, rsem0, wsem0)
    _issue(1, g1, rb1, wb1, gsem1, rsem1, wsem1)

    def pair(i, carry):
        k0 = 2 * i
        k1 = 2 * i + 1
        _drain(k0, g0, rb0, wb0, gsem0, rsem0, wsem0, ssem0)
        _drain(k1, g1, rb1, wb1, gsem1, rsem1, wsem1, ssem1)
        _swait(g0, rb0, ssem0)

        @pl.when(i + 1 < NPAIR)
        def _():
            _issue(k0 + 2, g0, rb0, wb0, gsem0, rsem0, wsem0)

        _swait(g1, rb1, ssem1)

        @pl.when(i + 1 < NPAIR)
        def _():
            _issue(k1 + 2, g1, rb1, wb1, gsem1, rsem1, wsem1)

        return carry

    lax.fori_loop(0, NPAIR, pair, 0)
    plsc.subcore_barrier()

    for i in range((NOUT_CHUNKS + NSUB - 1) // NSUB):
        g = s + i * NSUB

        @pl.when(g < NOUT_CHUNKS)
        def _():
            r0 = g * OUT_CHUNK
            pltpu.sync_copy(acc.at[pl.ds(r0, OUT_CHUNK)], g0.at[pl.ds(0, OUT_CHUNK)])
            pltpu.sync_copy(g0.at[pl.ds(0, OUT_CHUNK)], out_hbm.at[c, pl.ds(r0, OUT_CHUNK)])


_spmm_call = pl.kernel(
    _spmm_body,
    out_type=jax.ShapeDtypeStruct((NCORES, N, D), jnp.float32),
    mesh=_MESH,
    scratch_types=[
        pltpu.VMEM((PCH, CHUNK), jnp.int32),
        pltpu.VMEM((1, CHUNK), jnp.int32),
        pltpu.VMEM((1, CHUNK), jnp.int32),
        pltpu.VMEM((1, CHUNK), jnp.float32),
        pltpu.VMEM((1, CHUNK), jnp.float32),
        pltpu.VMEM((CHUNK, D), jnp.float32),
        pltpu.VMEM((CHUNK, D), jnp.float32),
        pltpu.VMEM_SHARED((N, D), jnp.float32),
        pltpu.SemaphoreType.DMA,
        pltpu.SemaphoreType.DMA,
        pltpu.SemaphoreType.DMA,
        pltpu.SemaphoreType.DMA,
        pltpu.SemaphoreType.DMA,
        pltpu.SemaphoreType.DMA,
        pltpu.SemaphoreType.DMA,
        pltpu.SemaphoreType.DMA,
    ],
)


def _pad_edges(a):
    aw = a.reshape(NW, EPW)
    ap = jnp.pad(aw, ((0, 0), (0, EPW_PAD - EPW)))
    return ap.reshape(NW, PCH, CHUNK)


def _spmm(x, col3, row3, w3):
    return _spmm_call(x, col3, row3, w3)


_ROWS_BLK = 2000


def _combine_body(p_ref, o_ref):
    o_ref[...] = p_ref[0] + p_ref[1]


def _combine(p):
    return pl.pallas_call(
        _combine_body,
        grid=(N // _ROWS_BLK,),
        in_specs=[pl.BlockSpec((NCORES, _ROWS_BLK, D), lambda i: (0, i, 0))],
        out_specs=pl.BlockSpec((_ROWS_BLK, D), lambda i: (i, 0)),
        out_shape=jax.ShapeDtypeStruct((N, D), jnp.float32),
    )(p)


def _layer_body(a_ref, b_ref, p_ref, wa_ref, wb_ref, wc_ref, o_ref):
    cc = p_ref[0] + p_ref[1]
    acc = jnp.dot(a_ref[...], wa_ref[...], preferred_element_type=jnp.float32)
    acc += jnp.dot(b_ref[...], wb_ref[...], preferred_element_type=jnp.float32)
    acc += jnp.dot(cc, wc_ref[...], preferred_element_type=jnp.float32)
    o_ref[...] = jnp.maximum(acc * SCALE, 0.0)


def _layer(a, b, p, wa, wb, wc):
    wspec = pl.BlockSpec((D, D), lambda i: (0, 0))
    return pl.pallas_call(
        _layer_body,
        grid=(N // _ROWS_BLK,),
        in_specs=[
            pl.BlockSpec((_ROWS_BLK, D), lambda i: (i, 0)),
            pl.BlockSpec((_ROWS_BLK, D), lambda i: (i, 0)),
            pl.BlockSpec((NCORES, _ROWS_BLK, D), lambda i: (0, i, 0)),
            wspec, wspec, wspec,
        ],
        out_specs=pl.BlockSpec((_ROWS_BLK, D), lambda i: (i, 0)),
        out_shape=jax.ShapeDtypeStruct((N, D), jnp.float32),
    )(a, b, p, wa, wb, wc)


def _final_body(a_ref, b_ref, p_ref, wa_ref, wb_ref, wc_ref, wm_ref, o_ref):
    cc = p_ref[0] + p_ref[1]
    acc = jnp.dot(a_ref[...], wa_ref[...], preferred_element_type=jnp.float32)
    acc += jnp.dot(b_ref[...], wb_ref[...], preferred_element_type=jnp.float32)
    acc += jnp.dot(cc, wc_ref[...], preferred_element_type=jnp.float32)
    h = jnp.maximum(acc * SCALE, 0.0)
    o_ref[...] = jnp.dot(h, wm_ref[...], preferred_element_type=jnp.float32) * SCALE


def _final(a, b, p, wa, wb, wc, wm):
    wspec = pl.BlockSpec((D, D), lambda i: (0, 0))
    return pl.pallas_call(
        _final_body,
        grid=(N // _ROWS_BLK,),
        in_specs=[
            pl.BlockSpec((_ROWS_BLK, D), lambda i: (i, 0)),
            pl.BlockSpec((_ROWS_BLK, D), lambda i: (i, 0)),
            pl.BlockSpec((NCORES, _ROWS_BLK, D), lambda i: (0, i, 0)),
            wspec, wspec, wspec,
            pl.BlockSpec((D, D_OUT), lambda i: (0, 0)),
        ],
        out_specs=pl.BlockSpec((_ROWS_BLK, D_OUT), lambda i: (i, 0)),
        out_shape=jax.ShapeDtypeStruct((N, D_OUT), jnp.float32),
    )(a, b, p, wa, wb, wc, wm)


def kernel(x, edge_index, edge_weight, W0_0, W0_1, W0_2, W1_0, W1_1, W1_2, Wmlp0):
    row3 = _pad_edges(edge_index[0])
    col3 = _pad_edges(edge_index[1])
    w3 = _pad_edges(edge_weight)
    p1 = _spmm(x, col3, row3, w3)
    z1 = _combine(p1)
    p2 = _spmm(z1, col3, row3, w3)
    y0 = _layer(x, z1, p2, W0_0, W0_1, W0_2)
    p3 = _spmm(y0, col3, row3, w3)
    t1 = _combine(p3)
    p4 = _spmm(t1, col3, row3, w3)
    return _final(y0, t1, p4, W1_0, W1_1, W1_2, Wmlp0.T)


# speedup vs baseline: 1.0361x; 1.0361x over previous
import functools
import math

import jax
import jax.numpy as jnp
from jax import lax
from jax.experimental import pallas as pl
from jax.experimental.pallas import tpu as pltpu
from jax.experimental.pallas import tpu_sc as plsc

N = 10000
E = 320000
D = 128
D_OUT = 64
NCORES = 2
NSUB = 16
NW = NCORES * NSUB
EPW = E // NW
CHUNK = 128
PCH = (EPW + CHUNK - 1) // CHUNK
PCH += PCH % 2
EPW_PAD = PCH * CHUNK
NPAIR = PCH // 2
OUT_CHUNK = 80
NOUT_CHUNKS = N // OUT_CHUNK
SCALE = 1.0 / math.sqrt(128.0)

_MESH = plsc.VectorSubcoreMesh(
    core_axis_name="c", subcore_axis_name="s", num_cores=NCORES, num_subcores=NSUB
)


def _spmm_body(x_hbm, col_hbm, row_hbm, w_hbm, out_hbm,
               colv, rb0, rb1, wb0, wb1, g0, g1, acc,
               gsem0, gsem1, rsem0, rsem1, wsem0, wsem1):
    c = lax.axis_index("c")
    s = lax.axis_index("s")
    wid = c * NSUB + s

    pltpu.sync_copy(col_hbm.at[wid], colv)

    def zero_row(e, carry):
        for j in range(D // 16):
            g0[e, pl.ds(j * 16, 16)] = jnp.zeros((16,), jnp.float32)
        return carry

    lax.fori_loop(0, OUT_CHUNK, zero_row, 0)
    for i in range((NOUT_CHUNKS + NSUB - 1) // NSUB):
        g = s + i * NSUB

        @pl.when(g < NOUT_CHUNKS)
        def _():
            pltpu.sync_copy(
                g0.at[pl.ds(0, OUT_CHUNK)], acc.at[pl.ds(g * OUT_CHUNK, OUT_CHUNK)]
            )

    plsc.subcore_barrier()

    def _scale(gbuf, wb):
        def scale_group(g, carry2):
            b16 = pl.multiple_of(g * 16, 16)
            wgrp = wb[0, pl.ds(b16, 16)]
            for e16 in range(16):
                we = wgrp[e16]
                r = b16 + e16
                for j in range(D // 16):
                    gbuf[r, pl.ds(j * 16, 16)] = gbuf[r, pl.ds(j * 16, 16)] * we
            return carry2

        lax.fori_loop(0, CHUNK // 16, scale_group, 0)

    def _issue(k, gbuf, rb, wb, gsem, rsem, wsem):
        pltpu.async_copy(x_hbm.at[colv.at[k]], gbuf, gsem)
        pltpu.async_copy(row_hbm.at[wid, pl.ds(k, 1)], rb, rsem)
        pltpu.async_copy(w_hbm.at[wid, pl.ds(k, 1)], wb, wsem)

    def _drain(k, gbuf, rb, wb, gsem, rsem, wsem):
        pltpu.make_async_copy(x_hbm.at[colv.at[k]], gbuf, gsem).wait()
        pltpu.make_async_copy(row_hbm.at[wid, pl.ds(k, 1)], rb, rsem).wait()
        pltpu.make_async_copy(w_hbm.at[wid, pl.ds(k, 1)], wb, wsem).wait()
        _scale(gbuf, wb)
        pltpu.sync_copy(gbuf, acc.at[rb.at[0]], add=True)

    _issue(0, g0, rb0, wb0, gsem0, rsem0, wsem0)

    def pair(i, carry):
        k0 = 2 * i
        k1 = 2 * i + 1
        _issue(k1, g1, rb1, wb1, gsem1, rsem1, wsem1)
        _drain(k0, g0, rb0, wb0, gsem0, rsem0, wsem0)

        @pl.when(i + 1 < NPAIR)
        def _():
            _issue(k0 + 2, g0, rb0, wb0, gsem0, rsem0, wsem0)

        _drain(k1, g1, rb1, wb1, gsem1, rsem1, wsem1)
        return carry

    lax.fori_loop(0, NPAIR, pair, 0)
    plsc.subcore_barrier()

    for i in range((NOUT_CHUNKS + NSUB - 1) // NSUB):
        g = s + i * NSUB

        @pl.when(g < NOUT_CHUNKS)
        def _():
            r0 = g * OUT_CHUNK
            pltpu.sync_copy(acc.at[pl.ds(r0, OUT_CHUNK)], g0.at[pl.ds(0, OUT_CHUNK)])
            pltpu.sync_copy(g0.at[pl.ds(0, OUT_CHUNK)], out_hbm.at[c, pl.ds(r0, OUT_CHUNK)])


_spmm_call = pl.kernel(
    _spmm_body,
    out_type=jax.ShapeDtypeStruct((NCORES, N, D), jnp.float32),
    mesh=_MESH,
    scratch_types=[
        pltpu.VMEM((PCH, CHUNK), jnp.int32),
        pltpu.VMEM((1, CHUNK), jnp.int32),
        pltpu.VMEM((1, CHUNK), jnp.int32),
        pltpu.VMEM((1, CHUNK), jnp.float32),
        pltpu.VMEM((1, CHUNK), jnp.float32),
        pltpu.VMEM((CHUNK, D), jnp.float32),
        pltpu.VMEM((CHUNK, D), jnp.float32),
        pltpu.VMEM_SHARED((N, D), jnp.float32),
        pltpu.SemaphoreType.DMA,
        pltpu.SemaphoreType.DMA,
        pltpu.SemaphoreType.DMA,
        pltpu.SemaphoreType.DMA,
        pltpu.SemaphoreType.DMA,
        pltpu.SemaphoreType.DMA,
    ],
)


def _pad_edges(a):
    aw = a.reshape(NW, EPW)
    ap = jnp.pad(aw, ((0, 0), (0, EPW_PAD - EPW)))
    return ap.reshape(NW, PCH, CHUNK)


def _spmm(x, col3, row3, w3):
    return _spmm_call(x, col3, row3, w3)


_ROWS_BLK = 2000


def _combine_body(p_ref, o_ref):
    o_ref[...] = p_ref[0] + p_ref[1]


def _combine(p):
    return pl.pallas_call(
        _combine_body,
        grid=(N // _ROWS_BLK,),
        in_specs=[pl.BlockSpec((NCORES, _ROWS_BLK, D), lambda i: (0, i, 0))],
        out_specs=pl.BlockSpec((_ROWS_BLK, D), lambda i: (i, 0)),
        out_shape=jax.ShapeDtypeStruct((N, D), jnp.float32),
    )(p)


def _layer_body(a_ref, b_ref, p_ref, wa_ref, wb_ref, wc_ref, o_ref):
    cc = p_ref[0] + p_ref[1]
    acc = jnp.dot(a_ref[...], wa_ref[...], preferred_element_type=jnp.float32)
    acc += jnp.dot(b_ref[...], wb_ref[...], preferred_element_type=jnp.float32)
    acc += jnp.dot(cc, wc_ref[...], preferred_element_type=jnp.float32)
    o_ref[...] = jnp.maximum(acc * SCALE, 0.0)


def _layer(a, b, p, wa, wb, wc):
    wspec = pl.BlockSpec((D, D), lambda i: (0, 0))
    return pl.pallas_call(
        _layer_body,
        grid=(N // _ROWS_BLK,),
        in_specs=[
            pl.BlockSpec((_ROWS_BLK, D), lambda i: (i, 0)),
            pl.BlockSpec((_ROWS_BLK, D), lambda i: (i, 0)),
            pl.BlockSpec((NCORES, _ROWS_BLK, D), lambda i: (0, i, 0)),
            wspec, wspec, wspec,
        ],
        out_specs=pl.BlockSpec((_ROWS_BLK, D), lambda i: (i, 0)),
        out_shape=jax.ShapeDtypeStruct((N, D), jnp.float32),
    )(a, b, p, wa, wb, wc)


def _final_body(a_ref, b_ref, p_ref, wa_ref, wb_ref, wc_ref, wm_ref, o_ref):
    cc = p_ref[0] + p_ref[1]
    acc = jnp.dot(a_ref[...], wa_ref[...], preferred_element_type=jnp.float32)
    acc += jnp.dot(b_ref[...], wb_ref[...], preferred_element_type=jnp.float32)
    acc += jnp.dot(cc, wc_ref[...], preferred_element_type=jnp.float32)
    h = jnp.maximum(acc * SCALE, 0.0)
    o_ref[...] = jnp.dot(h, wm_ref[...], preferred_element_type=jnp.float32) * SCALE


def _final(a, b, p, wa, wb, wc, wm):
    wspec = pl.BlockSpec((D, D), lambda i: (0, 0))
    return pl.pallas_call(
        _final_body,
        grid=(N // _ROWS_BLK,),
        in_specs=[
            pl.BlockSpec((_ROWS_BLK, D), lambda i: (i, 0)),
            pl.BlockSpec((_ROWS_BLK, D), lambda i: (i, 0)),
            pl.BlockSpec((NCORES, _ROWS_BLK, D), lambda i: (0, i, 0)),
            wspec, wspec, wspec,
            pl.BlockSpec((D, D_OUT), lambda i: (0, 0)),
        ],
        out_specs=pl.BlockSpec((_ROWS_BLK, D_OUT), lambda i: (i, 0)),
        out_shape=jax.ShapeDtypeStruct((N, D_OUT), jnp.float32),
    )(a, b, p, wa, wb, wc, wm)


def kernel(x, edge_index, edge_weight, W0_0, W0_1, W0_2, W1_0, W1_1, W1_2, Wmlp0):
    row3 = _pad_edges(edge_index[0])
    col3 = _pad_edges(edge_index[1])
    w3 = _pad_edges(edge_weight)
    p1 = _spmm(x, col3, row3, w3)
    z1 = _combine(p1)
    p2 = _spmm(z1, col3, row3, w3)
    y0 = _layer(x, z1, p2, W0_0, W0_1, W0_2)
    p3 = _spmm(y0, col3, row3, w3)
    t1 = _combine(p3)
    p4 = _spmm(t1, col3, row3, w3)
    return _final(y0, t1, p4, W1_0, W1_1, W1_2, Wmlp0.T)


# speedup vs baseline: 1.0390x; 1.0028x over previous
import functools
import math

import jax
import jax.numpy as jnp
from jax import lax
from jax.experimental import pallas as pl
from jax.experimental.pallas import tpu as pltpu
from jax.experimental.pallas import tpu_sc as plsc

N = 10000
E = 320000
D = 128
D_OUT = 64
NCORES = 2
NSUB = 16
NW = NCORES * NSUB
EPW = E // NW
CHUNK = 128
PCH = (EPW + CHUNK - 1) // CHUNK
PCH += PCH % 2
EPW_PAD = PCH * CHUNK
NPAIR = PCH // 2
OUT_CHUNK = 80
NOUT_CHUNKS = N // OUT_CHUNK
SCALE = 1.0 / math.sqrt(128.0)

_MESH = plsc.VectorSubcoreMesh(
    core_axis_name="c", subcore_axis_name="s", num_cores=NCORES, num_subcores=NSUB
)


def _spmm_body(x_hbm, col_hbm, row_hbm, w_hbm, out_hbm,
               colv, rb0, rb1, wb0, wb1, g0, g1, acc,
               gsem0, gsem1, gsem0b, gsem1b, rsem0, rsem1, wsem0, wsem1):
    c = lax.axis_index("c")
    s = lax.axis_index("s")
    wid = c * NSUB + s

    pltpu.sync_copy(col_hbm.at[wid], colv)

    def zero_row(e, carry):
        for j in range(D // 16):
            g0[e, pl.ds(j * 16, 16)] = jnp.zeros((16,), jnp.float32)
        return carry

    lax.fori_loop(0, OUT_CHUNK, zero_row, 0)
    for i in range((NOUT_CHUNKS + NSUB - 1) // NSUB):
        g = s + i * NSUB

        @pl.when(g < NOUT_CHUNKS)
        def _():
            pltpu.sync_copy(
                g0.at[pl.ds(0, OUT_CHUNK)], acc.at[pl.ds(g * OUT_CHUNK, OUT_CHUNK)]
            )

    plsc.subcore_barrier()

    def _scale(gbuf, wb):
        def scale_group(g, carry2):
            b16 = pl.multiple_of(g * 16, 16)
            wgrp = wb[0, pl.ds(b16, 16)]
            for e16 in range(16):
                we = wgrp[e16]
                r = b16 + e16
                for j in range(D // 16):
                    gbuf[r, pl.ds(j * 16, 16)] = gbuf[r, pl.ds(j * 16, 16)] * we
            return carry2

        lax.fori_loop(0, CHUNK // 16, scale_group, 0)

    H = CHUNK // 2

    def _issue(k, gbuf, rb, wb, gsem, gsemb, rsem, wsem):
        pltpu.async_copy(x_hbm.at[colv.at[k, pl.ds(0, H)]], gbuf.at[pl.ds(0, H)], gsem)
        pltpu.async_copy(x_hbm.at[colv.at[k, pl.ds(H, H)]], gbuf.at[pl.ds(H, H)], gsemb)
        pltpu.async_copy(row_hbm.at[wid, pl.ds(k, 1)], rb, rsem)
        pltpu.async_copy(w_hbm.at[wid, pl.ds(k, 1)], wb, wsem)

    def _drain(k, gbuf, rb, wb, gsem, gsemb, rsem, wsem):
        pltpu.make_async_copy(x_hbm.at[colv.at[k, pl.ds(0, H)]], gbuf.at[pl.ds(0, H)], gsem).wait()
        pltpu.make_async_copy(x_hbm.at[colv.at[k, pl.ds(H, H)]], gbuf.at[pl.ds(H, H)], gsemb).wait()
        pltpu.make_async_copy(row_hbm.at[wid, pl.ds(k, 1)], rb, rsem).wait()
        pltpu.make_async_copy(w_hbm.at[wid, pl.ds(k, 1)], wb, wsem).wait()
        _scale(gbuf, wb)
        pltpu.sync_copy(gbuf, acc.at[rb.at[0]], add=True)

    _issue(0, g0, rb0, wb0, gsem0, gsem0b, rsem0, wsem0)

    def pair(i, carry):
        k0 = 2 * i
        k1 = 2 * i + 1
        _issue(k1, g1, rb1, wb1, gsem1, gsem1b, rsem1, wsem1)
        _drain(k0, g0, rb0, wb0, gsem0, gsem0b, rsem0, wsem0)

        @pl.when(i + 1 < NPAIR)
        def _():
            _issue(k0 + 2, g0, rb0, wb0, gsem0, gsem0b, rsem0, wsem0)

        _drain(k1, g1, rb1, wb1, gsem1, gsem1b, rsem1, wsem1)
        return carry

    lax.fori_loop(0, NPAIR, pair, 0)
    plsc.subcore_barrier()

    for i in range((NOUT_CHUNKS + NSUB - 1) // NSUB):
        g = s + i * NSUB

        @pl.when(g < NOUT_CHUNKS)
        def _():
            r0 = g * OUT_CHUNK
            pltpu.sync_copy(acc.at[pl.ds(r0, OUT_CHUNK)], g0.at[pl.ds(0, OUT_CHUNK)])
            pltpu.sync_copy(g0.at[pl.ds(0, OUT_CHUNK)], out_hbm.at[c, pl.ds(r0, OUT_CHUNK)])


_spmm_call = pl.kernel(
    _spmm_body,
    out_type=jax.ShapeDtypeStruct((NCORES, N, D), jnp.float32),
    mesh=_MESH,
    scratch_types=[
        pltpu.VMEM((PCH, CHUNK), jnp.int32),
        pltpu.VMEM((1, CHUNK), jnp.int32),
        pltpu.VMEM((1, CHUNK), jnp.int32),
        pltpu.VMEM((1, CHUNK), jnp.float32),
        pltpu.VMEM((1, CHUNK), jnp.float32),
        pltpu.VMEM((CHUNK, D), jnp.float32),
        pltpu.VMEM((CHUNK, D), jnp.float32),
        pltpu.VMEM_SHARED((N, D), jnp.float32),
        pltpu.SemaphoreType.DMA,
        pltpu.SemaphoreType.DMA,
        pltpu.SemaphoreType.DMA,
        pltpu.SemaphoreType.DMA,
        pltpu.SemaphoreType.DMA,
        pltpu.SemaphoreType.DMA,
        pltpu.SemaphoreType.DMA,
        pltpu.SemaphoreType.DMA,
    ],
)


def _pad_edges(a):
    aw = a.reshape(NW, EPW)
    ap = jnp.pad(aw, ((0, 0), (0, EPW_PAD - EPW)))
    return ap.reshape(NW, PCH, CHUNK)


def _spmm(x, col3, row3, w3):
    return _spmm_call(x, col3, row3, w3)


_ROWS_BLK = 2000


def _combine_body(p_ref, o_ref):
    o_ref[...] = p_ref[0] + p_ref[1]


def _combine(p):
    return pl.pallas_call(
        _combine_body,
        grid=(N // _ROWS_BLK,),
        in_specs=[pl.BlockSpec((NCORES, _ROWS_BLK, D), lambda i: (0, i, 0))],
        out_specs=pl.BlockSpec((_ROWS_BLK, D), lambda i: (i, 0)),
        out_shape=jax.ShapeDtypeStruct((N, D), jnp.float32),
    )(p)


def _layer_body(a_ref, b_ref, p_ref, wa_ref, wb_ref, wc_ref, o_ref):
    cc = p_ref[0] + p_ref[1]
    acc = jnp.dot(a_ref[...], wa_ref[...], preferred_element_type=jnp.float32)
    acc += jnp.dot(b_ref[...], wb_ref[...], preferred_element_type=jnp.float32)
    acc += jnp.dot(cc, wc_ref[...], preferred_element_type=jnp.float32)
    o_ref[...] = jnp.maximum(acc * SCALE, 0.0)


def _layer(a, b, p, wa, wb, wc):
    wspec = pl.BlockSpec((D, D), lambda i: (0, 0))
    return pl.pallas_call(
        _layer_body,
        grid=(N // _ROWS_BLK,),
        in_specs=[
            pl.BlockSpec((_ROWS_BLK, D), lambda i: (i, 0)),
            pl.BlockSpec((_ROWS_BLK, D), lambda i: (i, 0)),
            pl.BlockSpec((NCORES, _ROWS_BLK, D), lambda i: (0, i, 0)),
            wspec, wspec, wspec,
        ],
        out_specs=pl.BlockSpec((_ROWS_BLK, D), lambda i: (i, 0)),
        out_shape=jax.ShapeDtypeStruct((N, D), jnp.float32),
    )(a, b, p, wa, wb, wc)


def _final_body(a_ref, b_ref, p_ref, wa_ref, wb_ref, wc_ref, wm_ref, o_ref):
    cc = p_ref[0] + p_ref[1]
    acc = jnp.dot(a_ref[...], wa_ref[...], preferred_element_type=jnp.float32)
    acc += jnp.dot(b_ref[...], wb_ref[...], preferred_element_type=jnp.float32)
    acc += jnp.dot(cc, wc_ref[...], preferred_element_type=jnp.float32)
    h = jnp.maximum(acc * SCALE, 0.0)
    o_ref[...] = jnp.dot(h, wm_ref[...], preferred_element_type=jnp.float32) * SCALE


def _final(a, b, p, wa, wb, wc, wm):
    wspec = pl.BlockSpec((D, D), lambda i: (0, 0))
    return pl.pallas_call(
        _final_body,
        grid=(N // _ROWS_BLK,),
        in_specs=[
            pl.BlockSpec((_ROWS_BLK, D), lambda i: (i, 0)),
            pl.BlockSpec((_ROWS_BLK, D), lambda i: (i, 0)),
            pl.BlockSpec((NCORES, _ROWS_BLK, D), lambda i: (0, i, 0)),
            wspec, wspec, wspec,
            pl.BlockSpec((D, D_OUT), lambda i: (0, 0)),
        ],
        out_specs=pl.BlockSpec((_ROWS_BLK, D_OUT), lambda i: (i, 0)),
        out_shape=jax.ShapeDtypeStruct((N, D_OUT), jnp.float32),
    )(a, b, p, wa, wb, wc, wm)


def kernel(x, edge_index, edge_weight, W0_0, W0_1, W0_2, W1_0, W1_1, W1_2, Wmlp0):
    row3 = _pad_edges(edge_index[0])
    col3 = _pad_edges(edge_index[1])
    w3 = _pad_edges(edge_weight)
    p1 = _spmm(x, col3, row3, w3)
    z1 = _combine(p1)
    p2 = _spmm(z1, col3, row3, w3)
    y0 = _layer(x, z1, p2, W0_0, W0_1, W0_2)
    p3 = _spmm(y0, col3, row3, w3)
    t1 = _combine(p3)
    p4 = _spmm(t1, col3, row3, w3)
    return _final(y0, t1, p4, W1_0, W1_1, W1_2, Wmlp0.T)
